# Initial kernel scaffold; baseline (speedup 1.0000x reference)
import jax
import jax.numpy as jnp
from jax.experimental import pallas as pl


def kernel(input, input_timestamp, edge_index, emb, W1, b1, W2, b2):
    raise NotImplementedError("write your pallas kernel here")


# speedup vs baseline: 10.9315x; 10.9315x over previous
import functools

import jax
import jax.numpy as jnp
from jax import lax
from jax.experimental import pallas as pl
from jax.experimental.pallas import tpu as pltpu
from jax.experimental.pallas import tpu_sc as plsc

NTOK = 10000
NPAD = 10240
STRIPE = NPAD // 16
NINP = 256
NHID = 512
NE = 160000
PADROW = 10016
EP = 172032
NCH = EP // (16 * 128)
DEG_EP = 161792
NCH_DEG = DEG_EP // (16 * 128)
BLK = 1024
_MESH = plsc.VectorSubcoreMesh(core_axis_name="c", subcore_axis_name="s")
_SC_PARAMS = pltpu.CompilerParams(needs_layout_passes=False)


def _deg_body(cols_hbm, ones_hbm, init_hbm, deg_hbm, cbuf, ones_v, acc):
    c = lax.axis_index("c")
    s = lax.axis_index("s")
    pltpu.sync_copy(cols_hbm.at[s], cbuf)
    pltpu.sync_copy(ones_hbm, ones_v)
    pltpu.sync_copy(init_hbm, acc.at[pl.ds(s * STRIPE, STRIPE)])
    plsc.subcore_barrier()

    def body(j, carry):
        pltpu.sync_copy(ones_v, acc.at[cbuf.at[j]], add=True)
        return carry

    lax.fori_loop(0, NCH_DEG, body, 0)
    plsc.subcore_barrier()

    @pl.when(c == 0)
    def _():
        pltpu.sync_copy(acc.at[pl.ds(s * STRIPE, STRIPE)],
                        deg_hbm.at[pl.ds(s * STRIPE, STRIPE)])


_deg = pl.kernel(
    _deg_body,
    mesh=_MESH,
    out_type=jax.ShapeDtypeStruct((NPAD,), jnp.float32),
    scratch_types=[
        pltpu.VMEM((NCH_DEG, 128), jnp.int32),
        pltpu.VMEM((128,), jnp.float32),
        pltpu.VMEM_SHARED((NPAD,), jnp.float32),
    ],
    compiler_params=_SC_PARAMS,
)


def _prop_body(rows_hbm, cols_hbm, xs0_hbm, xs1_hbm, zeros_hbm, out_hbm,
               rbuf, cbuf, gbuf, acc, sem):
    c = lax.axis_index("c")
    s = lax.axis_index("s")
    pltpu.sync_copy(rows_hbm.at[s], rbuf)
    pltpu.sync_copy(cols_hbm.at[s], cbuf)
    pltpu.sync_copy(zeros_hbm, acc.at[pl.ds(s * STRIPE, STRIPE)])
    plsc.subcore_barrier()

    def run(xs_hbm):
        def body(j, carry):
            pltpu.async_copy(xs_hbm.at[rbuf.at[j]], gbuf, sem).wait()
            pltpu.sync_copy(gbuf, acc.at[cbuf.at[j]], add=True)
            return carry

        lax.fori_loop(0, NCH, body, 0)

    @pl.when(c == 0)
    def _():
        run(xs0_hbm)

    @pl.when(c == 1)
    def _():
        run(xs1_hbm)

    plsc.subcore_barrier()

    @pl.when(c == 0)
    def _():
        pltpu.sync_copy(acc.at[pl.ds(s * STRIPE, STRIPE)],
                        out_hbm.at[pl.ds(s * STRIPE, STRIPE), pl.ds(0, 128)])

    @pl.when(c == 1)
    def _():
        pltpu.sync_copy(acc.at[pl.ds(s * STRIPE, STRIPE)],
                        out_hbm.at[pl.ds(s * STRIPE, STRIPE), pl.ds(128, 128)])


_prop = pl.kernel(
    _prop_body,
    mesh=_MESH,
    out_type=jax.ShapeDtypeStruct((NPAD, NINP), jnp.float32),
    scratch_types=[
        pltpu.VMEM((NCH, 128), jnp.int32),
        pltpu.VMEM((NCH, 128), jnp.int32),
        pltpu.VMEM((128, 128), jnp.float32),
        pltpu.VMEM_SHARED((NPAD, 128), jnp.float32),
        pltpu.SemaphoreType.DMA,
    ],
    compiler_params=_SC_PARAMS,
)


def _lookup_body(tok_hbm, s2_hbm, dinv_hbm, b2_hbm, out_hbm,
                 ibuf, dtab, b2v, gbuf, obuf, sem):
    c = lax.axis_index("c")
    s = lax.axis_index("s")
    w = s * 2 + c
    pltpu.sync_copy(tok_hbm.at[w], ibuf)
    pltpu.sync_copy(dinv_hbm, dtab)
    pltpu.sync_copy(b2_hbm, b2v)
    pltpu.async_copy(s2_hbm.at[ibuf], gbuf, sem).wait()
    dv0 = plsc.load_gather(dtab, [ibuf[pl.ds(0, 16)]])
    dv1 = plsc.load_gather(dtab, [ibuf[pl.ds(16, 16)]])
    riota = jnp.arange(16, dtype=jnp.int32)

    def body(f, carry):
        fs = jnp.full((16,), f, dtype=jnp.int32)
        bv = plsc.load_gather(b2v, [fs])
        v0 = plsc.load_gather(gbuf, [riota, fs]) * dv0 + bv
        v1 = plsc.load_gather(gbuf, [riota + 16, fs]) * dv1 + bv
        plsc.store_scatter(obuf, [riota, fs], v0)
        plsc.store_scatter(obuf, [riota + 16, fs], v1)
        return carry

    lax.fori_loop(0, NINP, body, 0)
    pltpu.sync_copy(obuf, out_hbm.at[pl.ds(w * 32, 32)])


_lookup = pl.kernel(
    _lookup_body,
    mesh=_MESH,
    out_type=jax.ShapeDtypeStruct((1024, NINP), jnp.float32),
    scratch_types=[
        pltpu.VMEM((32,), jnp.int32),
        pltpu.VMEM((NPAD,), jnp.float32),
        pltpu.VMEM((NINP,), jnp.float32),
        pltpu.VMEM((32, NINP), jnp.float32),
        pltpu.VMEM((32, NINP), jnp.float32),
        pltpu.SemaphoreType.DMA,
    ],
    compiler_params=_SC_PARAMS,
)


def _scale_body(deg_ref, emb_ref, dinv_ref, xs0_ref, xs1_ref):
    deg = deg_ref[...]
    dv = jnp.where(deg > 0, lax.rsqrt(deg), 0.0)
    dinv_ref[...] = dv
    e = emb_ref[...]
    xs0_ref[...] = e[:, :128] * dv
    xs1_ref[...] = e[:, 128:] * dv


_scale = pl.pallas_call(
    _scale_body,
    grid=(NPAD // BLK,),
    in_specs=[
        pl.BlockSpec((BLK, 1), lambda i: (i, 0)),
        pl.BlockSpec((BLK, NINP), lambda i: (i, 0)),
    ],
    out_specs=[
        pl.BlockSpec((BLK, 1), lambda i: (i, 0)),
        pl.BlockSpec((BLK, 128), lambda i: (i, 0)),
        pl.BlockSpec((BLK, 128), lambda i: (i, 0)),
    ],
    out_shape=[
        jax.ShapeDtypeStruct((NPAD, 1), jnp.float32),
        jax.ShapeDtypeStruct((NPAD, 128), jnp.float32),
        jax.ShapeDtypeStruct((NPAD, 128), jnp.float32),
    ],
)


def _mid_body(s1_ref, dinv_ref, w1_ref, b1_ref, w2_ref, xs20_ref, xs21_ref):
    i = pl.program_id(0)
    dv = dinv_ref[...]
    u = s1_ref[...] * dv
    h = jnp.dot(u, w1_ref[...], preferred_element_type=jnp.float32,
                precision=lax.Precision.HIGHEST) + b1_ref[...]
    y = jnp.dot(h, w2_ref[...], preferred_element_type=jnp.float32,
                precision=lax.Precision.HIGHEST) * dv
    rid = i * BLK + lax.broadcasted_iota(jnp.int32, (BLK, 1), 0)
    y = jnp.where(rid < NTOK, y, 0.0)
    xs20_ref[...] = y[:, :128]
    xs21_ref[...] = y[:, 128:]


_mid = pl.pallas_call(
    _mid_body,
    grid=(NPAD // BLK,),
    in_specs=[
        pl.BlockSpec((BLK, NINP), lambda i: (i, 0)),
        pl.BlockSpec((BLK, 1), lambda i: (i, 0)),
        pl.BlockSpec((NINP, NHID), lambda i: (0, 0)),
        pl.BlockSpec((1, NHID), lambda i: (0, 0)),
        pl.BlockSpec((NHID, NINP), lambda i: (0, 0)),
    ],
    out_specs=[
        pl.BlockSpec((BLK, 128), lambda i: (i, 0)),
        pl.BlockSpec((BLK, 128), lambda i: (i, 0)),
    ],
    out_shape=[
        jax.ShapeDtypeStruct((NPAD, 128), jnp.float32),
        jax.ShapeDtypeStruct((NPAD, 128), jnp.float32),
    ],
)


def kernel(input, input_timestamp, edge_index, emb, W1, b1, W2, b2):
    f32, i32 = jnp.float32, jnp.int32
    ei = edge_index.astype(i32)
    loops = jnp.arange(NTOK, dtype=i32)
    rows = jnp.concatenate([ei[0], loops,
                            jnp.full((EP - NE - NTOK,), PADROW, i32)])
    cols = jnp.concatenate([ei[1], loops,
                            jnp.zeros((EP - NE - NTOK,), i32)])
    rows_h = rows.reshape(16, NCH, 128)
    cols_h = cols.reshape(16, NCH, 128)
    cols_deg = jnp.concatenate(
        [ei[1], jnp.full((DEG_EP - NE,), PADROW, i32)]).reshape(16, NCH_DEG, 128)
    ones128 = jnp.ones((128,), f32)
    init640 = jnp.ones((STRIPE,), f32)
    zeros640 = jnp.zeros((STRIPE, 128), f32)
    emb_pad = jnp.pad(emb, ((0, NPAD - NTOK), (0, 0)))

    deg = _deg(cols_deg, ones128, init640)
    dinv, xs0, xs1 = _scale(deg.reshape(NPAD, 1), emb_pad)
    s1 = _prop(rows_h, cols_h, xs0, xs1, zeros640)
    xs20, xs21 = _mid(s1, dinv, W1, b1.reshape(1, NHID), W2)
    s2 = _prop(rows_h, cols_h, xs20, xs21, zeros640)

    flat = input.reshape(-1).astype(i32)
    tok = jnp.concatenate(
        [flat, jnp.zeros((1024 - flat.shape[0],), i32)]).reshape(32, 32)
    out = _lookup(tok, s2, dinv.reshape(-1), b2)
    return out[:flat.shape[0]].reshape(input.shape[0], input.shape[1], NINP)


# speedup vs baseline: 12.2349x; 1.1192x over previous
import functools

import jax
import jax.numpy as jnp
from jax import lax
from jax.experimental import pallas as pl
from jax.experimental.pallas import tpu as pltpu
from jax.experimental.pallas import tpu_sc as plsc

NTOK = 10000
NPAD = 10240
STRIPE = NPAD // 16
NINP = 256
NHID = 512
NE = 160000
PADROW = 10016
EP = 172032
NCH = EP // (16 * 128)
MCH = 28
DEG_EP = 161792
NCH_DEG = DEG_EP // (16 * 128)
BLK = 1024
_MESH = plsc.VectorSubcoreMesh(core_axis_name="c", subcore_axis_name="s")
_SC_PARAMS = pltpu.CompilerParams(needs_layout_passes=False)


def _deg_body(cols_hbm, ones_hbm, init_hbm, deg_hbm, cbuf, ones_v, acc):
    c = lax.axis_index("c")
    s = lax.axis_index("s")
    pltpu.sync_copy(cols_hbm.at[s], cbuf)
    pltpu.sync_copy(ones_hbm, ones_v)
    pltpu.sync_copy(init_hbm, acc.at[pl.ds(s * STRIPE, STRIPE)])
    plsc.subcore_barrier()

    def body(j, carry):
        pltpu.sync_copy(ones_v, acc.at[cbuf.at[j]], add=True)
        return carry

    lax.fori_loop(0, NCH_DEG, body, 0)
    plsc.subcore_barrier()

    @pl.when(c == 0)
    def _():
        pltpu.sync_copy(acc.at[pl.ds(s * STRIPE, STRIPE)],
                        deg_hbm.at[pl.ds(s * STRIPE, STRIPE)])


_deg = pl.kernel(
    _deg_body,
    mesh=_MESH,
    out_type=jax.ShapeDtypeStruct((NPAD,), jnp.float32),
    scratch_types=[
        pltpu.VMEM((NCH_DEG, 128), jnp.int32),
        pltpu.VMEM((128,), jnp.float32),
        pltpu.VMEM_SHARED((NPAD,), jnp.float32),
    ],
    compiler_params=_SC_PARAMS,
)


def _prop_body(rows_hbm, cols_hbm, xs0_hbm, xs1_hbm, zeros_hbm, out_hbm,
               rbuf, cbuf, gbufa, gbufb, acc, sema, semb):
    c = lax.axis_index("c")
    s = lax.axis_index("s")
    pltpu.sync_copy(zeros_hbm, acc.at[pl.ds(s * STRIPE, STRIPE)])
    plsc.subcore_barrier()

    def run(xs_hbm):
        def macro(m, carry):
            pltpu.sync_copy(rows_hbm.at[s, m], rbuf)
            pltpu.sync_copy(cols_hbm.at[s, m], cbuf)
            pltpu.async_copy(xs_hbm.at[rbuf.at[0]], gbufa, sema)
            pltpu.async_copy(xs_hbm.at[rbuf.at[1]], gbufb, semb)

            def body(i, carry2):
                j0 = 2 * i
                pltpu.make_async_copy(xs_hbm.at[rbuf.at[j0]], gbufa,
                                      sema).wait()
                pltpu.sync_copy(gbufa, acc.at[cbuf.at[j0]], add=True)

                @pl.when(i < MCH // 2 - 1)
                def _():
                    pltpu.async_copy(xs_hbm.at[rbuf.at[j0 + 2]], gbufa, sema)

                pltpu.make_async_copy(xs_hbm.at[rbuf.at[j0]], gbufb,
                                      semb).wait()
                pltpu.sync_copy(gbufb, acc.at[cbuf.at[j0 + 1]], add=True)

                @pl.when(i < MCH // 2 - 1)
                def _():
                    pltpu.async_copy(xs_hbm.at[rbuf.at[j0 + 3]], gbufb, semb)

                return carry2

            lax.fori_loop(0, MCH // 2, body, 0)
            return carry

        lax.fori_loop(0, NCH // MCH, macro, 0)

    @pl.when(c == 0)
    def _():
        run(xs0_hbm)

    @pl.when(c == 1)
    def _():
        run(xs1_hbm)

    plsc.subcore_barrier()

    @pl.when(c == 0)
    def _():
        pltpu.sync_copy(acc.at[pl.ds(s * STRIPE, STRIPE)],
                        out_hbm.at[pl.ds(s * STRIPE, STRIPE), pl.ds(0, 128)])

    @pl.when(c == 1)
    def _():
        pltpu.sync_copy(acc.at[pl.ds(s * STRIPE, STRIPE)],
                        out_hbm.at[pl.ds(s * STRIPE, STRIPE), pl.ds(128, 128)])


_prop = pl.kernel(
    _prop_body,
    mesh=_MESH,
    out_type=jax.ShapeDtypeStruct((NPAD, NINP), jnp.float32),
    scratch_types=[
        pltpu.VMEM((MCH, 128), jnp.int32),
        pltpu.VMEM((MCH, 128), jnp.int32),
        pltpu.VMEM((128, 128), jnp.float32),
        pltpu.VMEM((128, 128), jnp.float32),
        pltpu.VMEM_SHARED((NPAD, 128), jnp.float32),
        pltpu.SemaphoreType.DMA,
        pltpu.SemaphoreType.DMA,
    ],
    compiler_params=_SC_PARAMS,
)


def _lookup_body(tok_hbm, s2_hbm, dinv_hbm, b2_hbm, out_hbm,
                 ibuf, dtab, b2v, gbuf, obuf, sem):
    c = lax.axis_index("c")
    s = lax.axis_index("s")
    w = s * 2 + c
    pltpu.sync_copy(tok_hbm.at[w], ibuf)
    pltpu.sync_copy(dinv_hbm, dtab)
    pltpu.sync_copy(b2_hbm, b2v)
    pltpu.async_copy(s2_hbm.at[ibuf], gbuf, sem).wait()
    dv0 = plsc.load_gather(dtab, [ibuf[pl.ds(0, 16)]])
    dv1 = plsc.load_gather(dtab, [ibuf[pl.ds(16, 16)]])
    riota = jnp.arange(16, dtype=jnp.int32)

    def body(f, carry):
        fs = jnp.full((16,), f, dtype=jnp.int32)
        bv = plsc.load_gather(b2v, [fs])
        v0 = plsc.load_gather(gbuf, [riota, fs]) * dv0 + bv
        v1 = plsc.load_gather(gbuf, [riota + 16, fs]) * dv1 + bv
        plsc.store_scatter(obuf, [riota, fs], v0)
        plsc.store_scatter(obuf, [riota + 16, fs], v1)
        return carry

    lax.fori_loop(0, NINP, body, 0)
    pltpu.sync_copy(obuf, out_hbm.at[pl.ds(w * 32, 32)])


_lookup = pl.kernel(
    _lookup_body,
    mesh=_MESH,
    out_type=jax.ShapeDtypeStruct((1024, NINP), jnp.float32),
    scratch_types=[
        pltpu.VMEM((32,), jnp.int32),
        pltpu.VMEM((NPAD,), jnp.float32),
        pltpu.VMEM((NINP,), jnp.float32),
        pltpu.VMEM((32, NINP), jnp.float32),
        pltpu.VMEM((32, NINP), jnp.float32),
        pltpu.SemaphoreType.DMA,
    ],
    compiler_params=_SC_PARAMS,
)


def _scale_body(deg_ref, emb_ref, dinv_ref, xs0_ref, xs1_ref):
    deg = deg_ref[...]
    dv = jnp.where(deg > 0, lax.rsqrt(deg), 0.0)
    dinv_ref[...] = dv
    e = emb_ref[...]
    xs0_ref[...] = e[:, :128] * dv
    xs1_ref[...] = e[:, 128:] * dv


_scale = pl.pallas_call(
    _scale_body,
    grid=(NPAD // BLK,),
    in_specs=[
        pl.BlockSpec((BLK, 1), lambda i: (i, 0)),
        pl.BlockSpec((BLK, NINP), lambda i: (i, 0)),
    ],
    out_specs=[
        pl.BlockSpec((BLK, 1), lambda i: (i, 0)),
        pl.BlockSpec((BLK, 128), lambda i: (i, 0)),
        pl.BlockSpec((BLK, 128), lambda i: (i, 0)),
    ],
    out_shape=[
        jax.ShapeDtypeStruct((NPAD, 1), jnp.float32),
        jax.ShapeDtypeStruct((NPAD, 128), jnp.float32),
        jax.ShapeDtypeStruct((NPAD, 128), jnp.float32),
    ],
)


def _mid_body(s1_ref, dinv_ref, w1_ref, b1_ref, w2_ref, xs20_ref, xs21_ref):
    i = pl.program_id(0)
    dv = dinv_ref[...]
    u = s1_ref[...] * dv
    h = jnp.dot(u, w1_ref[...], preferred_element_type=jnp.float32,
                precision=lax.Precision.HIGHEST) + b1_ref[...]
    y = jnp.dot(h, w2_ref[...], preferred_element_type=jnp.float32,
                precision=lax.Precision.HIGHEST) * dv
    rid = i * BLK + lax.broadcasted_iota(jnp.int32, (BLK, 1), 0)
    y = jnp.where(rid < NTOK, y, 0.0)
    xs20_ref[...] = y[:, :128]
    xs21_ref[...] = y[:, 128:]


_mid = pl.pallas_call(
    _mid_body,
    grid=(NPAD // BLK,),
    in_specs=[
        pl.BlockSpec((BLK, NINP), lambda i: (i, 0)),
        pl.BlockSpec((BLK, 1), lambda i: (i, 0)),
        pl.BlockSpec((NINP, NHID), lambda i: (0, 0)),
        pl.BlockSpec((1, NHID), lambda i: (0, 0)),
        pl.BlockSpec((NHID, NINP), lambda i: (0, 0)),
    ],
    out_specs=[
        pl.BlockSpec((BLK, 128), lambda i: (i, 0)),
        pl.BlockSpec((BLK, 128), lambda i: (i, 0)),
    ],
    out_shape=[
        jax.ShapeDtypeStruct((NPAD, 128), jnp.float32),
        jax.ShapeDtypeStruct((NPAD, 128), jnp.float32),
    ],
)


def kernel(input, input_timestamp, edge_index, emb, W1, b1, W2, b2):
    f32, i32 = jnp.float32, jnp.int32
    ei = edge_index.astype(i32)
    loops = jnp.arange(NTOK, dtype=i32)
    rows = jnp.concatenate([ei[0], loops,
                            jnp.full((EP - NE - NTOK,), PADROW, i32)])
    cols = jnp.concatenate([ei[1], loops,
                            jnp.zeros((EP - NE - NTOK,), i32)])
    rows_h = rows.reshape(16, NCH // MCH, MCH, 128)
    cols_h = cols.reshape(16, NCH // MCH, MCH, 128)
    cols_deg = jnp.concatenate(
        [ei[1], jnp.full((DEG_EP - NE,), PADROW, i32)]).reshape(16, NCH_DEG, 128)
    ones128 = jnp.ones((128,), f32)
    init640 = jnp.ones((STRIPE,), f32)
    zeros640 = jnp.zeros((STRIPE, 128), f32)
    emb_pad = jnp.pad(emb, ((0, NPAD - NTOK), (0, 0)))

    deg = _deg(cols_deg, ones128, init640)
    dinv, xs0, xs1 = _scale(deg.reshape(NPAD, 1), emb_pad)
    s1 = _prop(rows_h, cols_h, xs0, xs1, zeros640)
    xs20, xs21 = _mid(s1, dinv, W1, b1.reshape(1, NHID), W2)
    s2 = _prop(rows_h, cols_h, xs20, xs21, zeros640)

    flat = input.reshape(-1).astype(i32)
    tok = jnp.concatenate(
        [flat, jnp.zeros((1024 - flat.shape[0],), i32)]).reshape(32, 32)
    out = _lookup(tok, s2, dinv.reshape(-1), b2)
    return out[:flat.shape[0]].reshape(input.shape[0], input.shape[1], NINP)


# speedup vs baseline: 13.0452x; 1.0662x over previous
import functools

import jax
import jax.numpy as jnp
from jax import lax
from jax.experimental import pallas as pl
from jax.experimental.pallas import tpu as pltpu
from jax.experimental.pallas import tpu_sc as plsc

NTOK = 10000
NPAD = 10240
STRIPE = NPAD // 16
NINP = 256
NHID = 512
NE = 160000
PADROW = 10016
EP = 172032
NCH = EP // (16 * 128)
MCH = 28
DEG_EP = 161792
NCH_DEG = DEG_EP // (16 * 128)
BLK = 1024
_MESH = plsc.VectorSubcoreMesh(core_axis_name="c", subcore_axis_name="s")
_SC_PARAMS = pltpu.CompilerParams(needs_layout_passes=False)


def _plan_body(colsdeg_hbm, rows_hbm, cols_hbm, tok_hbm, ones_hbm, init_hbm,
               zbm_hbm, deg_hbm, crows_hbm, ccols_hbm, counts_hbm,
               dbuf, rbuf, cbuf, tbuf, bmap, crv, ccv, ones_v, cntb, acc):
    c = lax.axis_index("c")
    s = lax.axis_index("s")
    pltpu.sync_copy(colsdeg_hbm.at[s], dbuf)
    pltpu.sync_copy(rows_hbm.at[s], rbuf)
    pltpu.sync_copy(cols_hbm.at[s], cbuf)
    pltpu.sync_copy(tok_hbm, tbuf)
    pltpu.sync_copy(zbm_hbm, bmap)
    pltpu.sync_copy(ones_hbm, ones_v)
    pltpu.sync_copy(init_hbm, acc.at[pl.ds(s * STRIPE, STRIPE)])
    plsc.subcore_barrier()

    one16 = jnp.full((16,), 1, jnp.int32)

    def tloop(k, carry):
        t16 = tbuf[pl.ds(k * 16, 16)]
        plsc.store_scatter(bmap, [t16], one16)
        return carry

    lax.fori_loop(0, 1024 // 16, tloop, 0)

    def dloop(j, carry):
        pltpu.sync_copy(ones_v, acc.at[dbuf.at[j]], add=True)
        return carry

    lax.fori_loop(0, NCH_DEG, dloop, 0)

    def floop(j, off):
        for k in range(8):
            c16 = cbuf[j, pl.ds(k * 16, 16)]
            r16 = rbuf[j, pl.ds(k * 16, 16)]
            m = plsc.load_gather(bmap, [c16]) > 0
            plsc.store_compressed(crv.at[pl.ds(off, 16)], r16, mask=m)
            plsc.store_compressed(ccv.at[pl.ds(off, 16)], c16, mask=m)
            off = off + jnp.sum(m.astype(jnp.int32))
        return off

    cnt = lax.fori_loop(0, NCH, floop, jnp.int32(0))
    padr = jnp.full((16,), PADROW, jnp.int32)
    padc = jnp.zeros((16,), jnp.int32)
    for k in range(8):
        crv[pl.ds(cnt + k * 16, 16)] = padr
        ccv[pl.ds(cnt + k * 16, 16)] = padc
    cntb[pl.ds(0, 16)] = jnp.full((16,), cnt, jnp.int32)
    plsc.subcore_barrier()

    @pl.when(c == 0)
    def _():
        pltpu.sync_copy(acc.at[pl.ds(s * STRIPE, STRIPE)],
                        deg_hbm.at[pl.ds(s * STRIPE, STRIPE)])
        pltpu.sync_copy(crv.at[pl.ds(0, 16 * NCH * 8)], crows_hbm.at[s])
        pltpu.sync_copy(ccv.at[pl.ds(0, 16 * NCH * 8)], ccols_hbm.at[s])
        pltpu.sync_copy(cntb, counts_hbm.at[s])


_plan = pl.kernel(
    _plan_body,
    mesh=_MESH,
    out_type=(
        jax.ShapeDtypeStruct((NPAD,), jnp.float32),
        jax.ShapeDtypeStruct((16, NCH * 128), jnp.int32),
        jax.ShapeDtypeStruct((16, NCH * 128), jnp.int32),
        jax.ShapeDtypeStruct((16, 16), jnp.int32),
    ),
    scratch_types=[
        pltpu.VMEM((NCH_DEG, 128), jnp.int32),
        pltpu.VMEM((NCH, 128), jnp.int32),
        pltpu.VMEM((NCH, 128), jnp.int32),
        pltpu.VMEM((1024,), jnp.int32),
        pltpu.VMEM((NPAD,), jnp.int32),
        pltpu.VMEM((NCH * 128 + 128,), jnp.int32),
        pltpu.VMEM((NCH * 128 + 128,), jnp.int32),
        pltpu.VMEM((128,), jnp.float32),
        pltpu.VMEM((16,), jnp.int32),
        pltpu.VMEM_SHARED((NPAD,), jnp.float32),
    ],
    compiler_params=_SC_PARAMS,
)


def _prop_body(rows_hbm, cols_hbm, xs0_hbm, xs1_hbm, zeros_hbm, out_hbm,
               rbuf, cbuf, gbufa, gbufb, acc, sema, semb):
    c = lax.axis_index("c")
    s = lax.axis_index("s")
    pltpu.sync_copy(zeros_hbm, acc.at[pl.ds(s * STRIPE, STRIPE)])
    plsc.subcore_barrier()

    def run(xs_hbm):
        def macro(m, carry):
            pltpu.sync_copy(rows_hbm.at[s, m], rbuf)
            pltpu.sync_copy(cols_hbm.at[s, m], cbuf)
            pltpu.async_copy(xs_hbm.at[rbuf.at[0]], gbufa, sema)
            pltpu.async_copy(xs_hbm.at[rbuf.at[1]], gbufb, semb)

            def body(i, carry2):
                j0 = 2 * i
                pltpu.make_async_copy(xs_hbm.at[rbuf.at[j0]], gbufa,
                                      sema).wait()
                pltpu.sync_copy(gbufa, acc.at[cbuf.at[j0]], add=True)

                @pl.when(i < MCH // 2 - 1)
                def _():
                    pltpu.async_copy(xs_hbm.at[rbuf.at[j0 + 2]], gbufa, sema)

                pltpu.make_async_copy(xs_hbm.at[rbuf.at[j0]], gbufb,
                                      semb).wait()
                pltpu.sync_copy(gbufb, acc.at[cbuf.at[j0 + 1]], add=True)

                @pl.when(i < MCH // 2 - 1)
                def _():
                    pltpu.async_copy(xs_hbm.at[rbuf.at[j0 + 3]], gbufb, semb)

                return carry2

            lax.fori_loop(0, MCH // 2, body, 0)
            return carry

        lax.fori_loop(0, NCH // MCH, macro, 0)

    @pl.when(c == 0)
    def _():
        run(xs0_hbm)

    @pl.when(c == 1)
    def _():
        run(xs1_hbm)

    plsc.subcore_barrier()

    @pl.when(c == 0)
    def _():
        pltpu.sync_copy(acc.at[pl.ds(s * STRIPE, STRIPE)],
                        out_hbm.at[pl.ds(s * STRIPE, STRIPE), pl.ds(0, 128)])

    @pl.when(c == 1)
    def _():
        pltpu.sync_copy(acc.at[pl.ds(s * STRIPE, STRIPE)],
                        out_hbm.at[pl.ds(s * STRIPE, STRIPE), pl.ds(128, 128)])


_prop = pl.kernel(
    _prop_body,
    mesh=_MESH,
    out_type=jax.ShapeDtypeStruct((NPAD, NINP), jnp.float32),
    scratch_types=[
        pltpu.VMEM((MCH, 128), jnp.int32),
        pltpu.VMEM((MCH, 128), jnp.int32),
        pltpu.VMEM((128, 128), jnp.float32),
        pltpu.VMEM((128, 128), jnp.float32),
        pltpu.VMEM_SHARED((NPAD, 128), jnp.float32),
        pltpu.SemaphoreType.DMA,
        pltpu.SemaphoreType.DMA,
    ],
    compiler_params=_SC_PARAMS,
)


def _prop_f_body(crows_hbm, ccols_hbm, counts_hbm, xs0_hbm, xs1_hbm,
                 zeros_hbm, out_hbm, rbuf, cbuf, cntv, gbuf, acc, sem):
    c = lax.axis_index("c")
    s = lax.axis_index("s")
    pltpu.sync_copy(crows_hbm.at[s], rbuf)
    pltpu.sync_copy(ccols_hbm.at[s], cbuf)
    pltpu.sync_copy(counts_hbm, cntv)
    pltpu.sync_copy(zeros_hbm, acc.at[pl.ds(s * STRIPE, STRIPE)])
    plsc.subcore_barrier()
    cnt = jnp.max(cntv[s, pl.ds(0, 16)])
    nch = lax.div(cnt + 127, jnp.int32(128))

    def run(xs_hbm):
        def body(j, carry):
            pltpu.async_copy(xs_hbm.at[rbuf.at[j]], gbuf, sem).wait()
            pltpu.sync_copy(gbuf, acc.at[cbuf.at[j]], add=True)
            return carry

        lax.fori_loop(0, nch, body, 0)

    @pl.when(c == 0)
    def _():
        run(xs0_hbm)

    @pl.when(c == 1)
    def _():
        run(xs1_hbm)

    plsc.subcore_barrier()

    @pl.when(c == 0)
    def _():
        pltpu.sync_copy(acc.at[pl.ds(s * STRIPE, STRIPE)],
                        out_hbm.at[pl.ds(s * STRIPE, STRIPE), pl.ds(0, 128)])

    @pl.when(c == 1)
    def _():
        pltpu.sync_copy(acc.at[pl.ds(s * STRIPE, STRIPE)],
                        out_hbm.at[pl.ds(s * STRIPE, STRIPE), pl.ds(128, 128)])


_prop_f = pl.kernel(
    _prop_f_body,
    mesh=_MESH,
    out_type=jax.ShapeDtypeStruct((NPAD, NINP), jnp.float32),
    scratch_types=[
        pltpu.VMEM((NCH, 128), jnp.int32),
        pltpu.VMEM((NCH, 128), jnp.int32),
        pltpu.VMEM((16, 16), jnp.int32),
        pltpu.VMEM((128, 128), jnp.float32),
        pltpu.VMEM_SHARED((NPAD, 128), jnp.float32),
        pltpu.SemaphoreType.DMA,
    ],
    compiler_params=_SC_PARAMS,
)


def _lookup_body(tok_hbm, s2_hbm, dinv_hbm, b2_hbm, out_hbm,
                 ibuf, dtab, b2v, gbuf, obuf, sem):
    c = lax.axis_index("c")
    s = lax.axis_index("s")
    w = s * 2 + c
    pltpu.sync_copy(tok_hbm.at[w], ibuf)
    pltpu.sync_copy(dinv_hbm, dtab)
    pltpu.sync_copy(b2_hbm, b2v)
    pltpu.async_copy(s2_hbm.at[ibuf], gbuf, sem).wait()
    dv0 = plsc.load_gather(dtab, [ibuf[pl.ds(0, 16)]])
    dv1 = plsc.load_gather(dtab, [ibuf[pl.ds(16, 16)]])
    riota = jnp.arange(16, dtype=jnp.int32)

    def body(f, carry):
        fs = jnp.full((16,), f, dtype=jnp.int32)
        bv = plsc.load_gather(b2v, [fs])
        v0 = plsc.load_gather(gbuf, [riota, fs]) * dv0 + bv
        v1 = plsc.load_gather(gbuf, [riota + 16, fs]) * dv1 + bv
        plsc.store_scatter(obuf, [riota, fs], v0)
        plsc.store_scatter(obuf, [riota + 16, fs], v1)
        return carry

    lax.fori_loop(0, NINP, body, 0)
    pltpu.sync_copy(obuf, out_hbm.at[pl.ds(w * 32, 32)])


_lookup = pl.kernel(
    _lookup_body,
    mesh=_MESH,
    out_type=jax.ShapeDtypeStruct((1024, NINP), jnp.float32),
    scratch_types=[
        pltpu.VMEM((32,), jnp.int32),
        pltpu.VMEM((NPAD,), jnp.float32),
        pltpu.VMEM((NINP,), jnp.float32),
        pltpu.VMEM((32, NINP), jnp.float32),
        pltpu.VMEM((32, NINP), jnp.float32),
        pltpu.SemaphoreType.DMA,
    ],
    compiler_params=_SC_PARAMS,
)


def _scale_body(deg_ref, emb_ref, dinv_ref, xs0_ref, xs1_ref):
    deg = deg_ref[...]
    dv = jnp.where(deg > 0, lax.rsqrt(deg), 0.0)
    dinv_ref[...] = dv
    e = emb_ref[...]
    xs0_ref[...] = e[:, :128] * dv
    xs1_ref[...] = e[:, 128:] * dv


_scale = pl.pallas_call(
    _scale_body,
    grid=(NPAD // BLK,),
    in_specs=[
        pl.BlockSpec((BLK, 1), lambda i: (i, 0)),
        pl.BlockSpec((BLK, NINP), lambda i: (i, 0)),
    ],
    out_specs=[
        pl.BlockSpec((BLK, 1), lambda i: (i, 0)),
        pl.BlockSpec((BLK, 128), lambda i: (i, 0)),
        pl.BlockSpec((BLK, 128), lambda i: (i, 0)),
    ],
    out_shape=[
        jax.ShapeDtypeStruct((NPAD, 1), jnp.float32),
        jax.ShapeDtypeStruct((NPAD, 128), jnp.float32),
        jax.ShapeDtypeStruct((NPAD, 128), jnp.float32),
    ],
)


def _mid_body(s1_ref, dinv_ref, w1_ref, b1_ref, w2_ref, xs20_ref, xs21_ref):
    i = pl.program_id(0)
    dv = dinv_ref[...]
    u = s1_ref[...] * dv
    h = jnp.dot(u, w1_ref[...], preferred_element_type=jnp.float32,
                precision=lax.Precision.HIGHEST) + b1_ref[...]
    y = jnp.dot(h, w2_ref[...], preferred_element_type=jnp.float32,
                precision=lax.Precision.HIGHEST) * dv
    rid = i * BLK + lax.broadcasted_iota(jnp.int32, (BLK, 1), 0)
    y = jnp.where(rid < NTOK, y, 0.0)
    xs20_ref[...] = y[:, :128]
    xs21_ref[...] = y[:, 128:]


_mid = pl.pallas_call(
    _mid_body,
    grid=(NPAD // BLK,),
    in_specs=[
        pl.BlockSpec((BLK, NINP), lambda i: (i, 0)),
        pl.BlockSpec((BLK, 1), lambda i: (i, 0)),
        pl.BlockSpec((NINP, NHID), lambda i: (0, 0)),
        pl.BlockSpec((1, NHID), lambda i: (0, 0)),
        pl.BlockSpec((NHID, NINP), lambda i: (0, 0)),
    ],
    out_specs=[
        pl.BlockSpec((BLK, 128), lambda i: (i, 0)),
        pl.BlockSpec((BLK, 128), lambda i: (i, 0)),
    ],
    out_shape=[
        jax.ShapeDtypeStruct((NPAD, 128), jnp.float32),
        jax.ShapeDtypeStruct((NPAD, 128), jnp.float32),
    ],
)


def kernel(input, input_timestamp, edge_index, emb, W1, b1, W2, b2):
    f32, i32 = jnp.float32, jnp.int32
    ei = edge_index.astype(i32)
    loops = jnp.arange(NTOK, dtype=i32)
    rows = jnp.concatenate([ei[0], loops,
                            jnp.full((EP - NE - NTOK,), PADROW, i32)])
    cols = jnp.concatenate([ei[1], loops,
                            jnp.zeros((EP - NE - NTOK,), i32)])
    rows_h = rows.reshape(16, NCH // MCH, MCH, 128)
    cols_h = cols.reshape(16, NCH // MCH, MCH, 128)
    rows_h3 = rows.reshape(16, NCH, 128)
    cols_h3 = cols.reshape(16, NCH, 128)
    cols_deg = jnp.concatenate(
        [ei[1], jnp.full((DEG_EP - NE,), PADROW, i32)]).reshape(16, NCH_DEG, 128)
    ones128 = jnp.ones((128,), f32)
    init640 = jnp.ones((STRIPE,), f32)
    zeros640 = jnp.zeros((STRIPE, 128), f32)
    zbm = jnp.zeros((NPAD,), i32)
    emb_pad = jnp.pad(emb, ((0, NPAD - NTOK), (0, 0)))
    flat = input.reshape(-1).astype(i32)
    tok = jnp.concatenate([flat, jnp.zeros((1024 - flat.shape[0],), i32)])

    deg, crows, ccols, counts = _plan(cols_deg, rows_h3, cols_h3, tok,
                                      ones128, init640, zbm)
    dinv, xs0, xs1 = _scale(deg.reshape(NPAD, 1), emb_pad)
    s1 = _prop(rows_h, cols_h, xs0, xs1, zeros640)
    xs20, xs21 = _mid(s1, dinv, W1, b1.reshape(1, NHID), W2)
    s2 = _prop_f(crows.reshape(16, NCH, 128), ccols.reshape(16, NCH, 128),
                 counts, xs20, xs21, zeros640)

    out = _lookup(tok.reshape(32, 32), s2, dinv.reshape(-1), b2)
    return out[:flat.shape[0]].reshape(input.shape[0], input.shape[1], NINP)


# speedup vs baseline: 16.2536x; 1.2459x over previous
import functools

import jax
import jax.numpy as jnp
from jax import lax
from jax.experimental import pallas as pl
from jax.experimental.pallas import tpu as pltpu
from jax.experimental.pallas import tpu_sc as plsc

NTOK = 10000
NPAD = 10240
STRIPE = NPAD // 16
NINP = 256
NHID = 512
NE = 160000
PADROW = 10016
EP = 172032
NCH = EP // (16 * 128)
MCH = 28
DEG_EP = 161792
NCH_DEG = DEG_EP // (16 * 128)
BLK = 1024
_MESH = plsc.VectorSubcoreMesh(core_axis_name="c", subcore_axis_name="s")
_SC_PARAMS = pltpu.CompilerParams(needs_layout_passes=False)


def _plan_body(colsdeg_hbm, rows_hbm, cols_hbm, tok_hbm, ones_hbm, init_hbm,
               zbm_hbm, deg_hbm, crows_hbm, ccols_hbm, counts_hbm,
               dbuf, rbuf, cbuf, tbuf, bmap, crv, ccv, ones_v, cntb, acc):
    c = lax.axis_index("c")
    s = lax.axis_index("s")
    pltpu.sync_copy(colsdeg_hbm.at[s], dbuf)
    pltpu.sync_copy(rows_hbm.at[s], rbuf)
    pltpu.sync_copy(cols_hbm.at[s], cbuf)
    pltpu.sync_copy(tok_hbm, tbuf)
    pltpu.sync_copy(zbm_hbm, bmap)
    pltpu.sync_copy(ones_hbm, ones_v)
    pltpu.sync_copy(init_hbm, acc.at[pl.ds(s * STRIPE, STRIPE)])
    plsc.subcore_barrier()

    one16 = jnp.full((16,), 1, jnp.int32)

    def tloop(k, carry):
        t16 = tbuf[pl.ds(k * 16, 16)]
        plsc.store_scatter(bmap, [t16], one16)
        return carry

    lax.fori_loop(0, 1024 // 16, tloop, 0)

    def dloop(j, carry):
        pltpu.sync_copy(ones_v, acc.at[dbuf.at[j]], add=True)
        return carry

    lax.fori_loop(0, NCH_DEG, dloop, 0)

    def floop(j, off):
        for k in range(8):
            c16 = cbuf[j, pl.ds(k * 16, 16)]
            r16 = rbuf[j, pl.ds(k * 16, 16)]
            m = plsc.load_gather(bmap, [c16]) > 0
            plsc.store_compressed(crv.at[pl.ds(off, 16)], r16, mask=m)
            plsc.store_compressed(ccv.at[pl.ds(off, 16)], c16, mask=m)
            off = off + jnp.sum(m.astype(jnp.int32))
        return off

    cnt = lax.fori_loop(0, NCH, floop, jnp.int32(0))
    padr = jnp.full((16,), PADROW, jnp.int32)
    padc = jnp.zeros((16,), jnp.int32)
    for k in range(8):
        crv[pl.ds(cnt + k * 16, 16)] = padr
        ccv[pl.ds(cnt + k * 16, 16)] = padc
    cntb[pl.ds(0, 16)] = jnp.full((16,), cnt, jnp.int32)
    plsc.subcore_barrier()

    @pl.when(c == 0)
    def _():
        pltpu.sync_copy(acc.at[pl.ds(s * STRIPE, STRIPE)],
                        deg_hbm.at[pl.ds(s * STRIPE, STRIPE)])
        pltpu.sync_copy(crv.at[pl.ds(0, 16 * NCH * 8)], crows_hbm.at[s])
        pltpu.sync_copy(ccv.at[pl.ds(0, 16 * NCH * 8)], ccols_hbm.at[s])
        pltpu.sync_copy(cntb, counts_hbm.at[s])


_plan = pl.kernel(
    _plan_body,
    mesh=_MESH,
    out_type=(
        jax.ShapeDtypeStruct((NPAD,), jnp.float32),
        jax.ShapeDtypeStruct((16, NCH * 128), jnp.int32),
        jax.ShapeDtypeStruct((16, NCH * 128), jnp.int32),
        jax.ShapeDtypeStruct((16, 16), jnp.int32),
    ),
    scratch_types=[
        pltpu.VMEM((NCH_DEG, 128), jnp.int32),
        pltpu.VMEM((NCH, 128), jnp.int32),
        pltpu.VMEM((NCH, 128), jnp.int32),
        pltpu.VMEM((1024,), jnp.int32),
        pltpu.VMEM((NPAD,), jnp.int32),
        pltpu.VMEM((NCH * 128 + 128,), jnp.int32),
        pltpu.VMEM((NCH * 128 + 128,), jnp.int32),
        pltpu.VMEM((128,), jnp.float32),
        pltpu.VMEM((16,), jnp.int32),
        pltpu.VMEM_SHARED((NPAD,), jnp.float32),
    ],
    compiler_params=_SC_PARAMS,
)


def _prop_body(rows_hbm, cols_hbm, xs0_hbm, xs1_hbm, zeros_hbm, out_hbm,
               rbuf, cbuf, gbufa, gbufb, acc, sema, semb):
    c = lax.axis_index("c")
    s = lax.axis_index("s")
    pltpu.sync_copy(zeros_hbm, acc.at[pl.ds(s * STRIPE, STRIPE)])
    plsc.subcore_barrier()

    def run(xs_hbm):
        def macro(m, carry):
            pltpu.sync_copy(rows_hbm.at[s, m], rbuf)
            pltpu.sync_copy(cols_hbm.at[s, m], cbuf)
            pltpu.async_copy(xs_hbm.at[rbuf.at[0]], gbufa, sema)
            pltpu.async_copy(xs_hbm.at[rbuf.at[1]], gbufb, semb)

            def body(i, carry2):
                j0 = 2 * i
                pltpu.make_async_copy(xs_hbm.at[rbuf.at[j0]], gbufa,
                                      sema).wait()
                pltpu.sync_copy(gbufa, acc.at[cbuf.at[j0]], add=True)

                @pl.when(i < MCH // 2 - 1)
                def _():
                    pltpu.async_copy(xs_hbm.at[rbuf.at[j0 + 2]], gbufa, sema)

                pltpu.make_async_copy(xs_hbm.at[rbuf.at[j0]], gbufb,
                                      semb).wait()
                pltpu.sync_copy(gbufb, acc.at[cbuf.at[j0 + 1]], add=True)

                @pl.when(i < MCH // 2 - 1)
                def _():
                    pltpu.async_copy(xs_hbm.at[rbuf.at[j0 + 3]], gbufb, semb)

                return carry2

            lax.fori_loop(0, MCH // 2, body, 0)
            return carry

        lax.fori_loop(0, NCH // MCH, macro, 0)

    @pl.when(c == 0)
    def _():
        run(xs0_hbm)

    @pl.when(c == 1)
    def _():
        run(xs1_hbm)

    plsc.subcore_barrier()

    @pl.when(c == 0)
    def _():
        pltpu.sync_copy(acc.at[pl.ds(s * STRIPE, STRIPE)],
                        out_hbm.at[pl.ds(s * STRIPE, STRIPE), pl.ds(0, 128)])

    @pl.when(c == 1)
    def _():
        pltpu.sync_copy(acc.at[pl.ds(s * STRIPE, STRIPE)],
                        out_hbm.at[pl.ds(s * STRIPE, STRIPE), pl.ds(128, 128)])


_prop = pl.kernel(
    _prop_body,
    mesh=_MESH,
    out_type=jax.ShapeDtypeStruct((NPAD, NINP), jnp.float32),
    scratch_types=[
        pltpu.VMEM((MCH, 128), jnp.int32),
        pltpu.VMEM((MCH, 128), jnp.int32),
        pltpu.VMEM((128, 128), jnp.float32),
        pltpu.VMEM((128, 128), jnp.float32),
        pltpu.VMEM_SHARED((NPAD, 128), jnp.float32),
        pltpu.SemaphoreType.DMA,
        pltpu.SemaphoreType.DMA,
    ],
    compiler_params=_SC_PARAMS,
)


def _prop_f_body(crows_hbm, ccols_hbm, counts_hbm, xs0_hbm, xs1_hbm,
                 zeros_hbm, out_hbm, rbuf, cbuf, cntv, gbufa, gbufb, acc,
                 sema, semb):
    c = lax.axis_index("c")
    s = lax.axis_index("s")
    pltpu.sync_copy(counts_hbm, cntv)
    pltpu.sync_copy(zeros_hbm, acc.at[pl.ds(s * STRIPE, STRIPE)])
    plsc.subcore_barrier()
    cnt = jnp.max(cntv[s, pl.ds(0, 16)])
    nch = lax.div(cnt + 127, jnp.int32(128))

    def run(xs_hbm):
        def macro(m, carry):
            pltpu.sync_copy(crows_hbm.at[s, m], rbuf)
            pltpu.sync_copy(ccols_hbm.at[s, m], cbuf)
            rem = jnp.minimum(nch - m * MCH, MCH)

            @pl.when(rem > 0)
            def _():
                pltpu.async_copy(xs_hbm.at[rbuf.at[0]], gbufa, sema)

            @pl.when(rem > 1)
            def _():
                pltpu.async_copy(xs_hbm.at[rbuf.at[1]], gbufb, semb)

            def body(i, carry2):
                j0 = 2 * i
                pltpu.make_async_copy(xs_hbm.at[rbuf.at[j0]], gbufa,
                                      sema).wait()
                pltpu.sync_copy(gbufa, acc.at[cbuf.at[j0]], add=True)

                @pl.when(j0 + 2 < rem)
                def _():
                    pltpu.async_copy(xs_hbm.at[rbuf.at[j0 + 2]], gbufa, sema)

                @pl.when(j0 + 1 < rem)
                def _():
                    pltpu.make_async_copy(xs_hbm.at[rbuf.at[j0]], gbufb,
                                          semb).wait()
                    pltpu.sync_copy(gbufb, acc.at[cbuf.at[j0 + 1]], add=True)

                    @pl.when(j0 + 3 < rem)
                    def _():
                        pltpu.async_copy(xs_hbm.at[rbuf.at[j0 + 3]], gbufb,
                                         semb)

                return carry2

            lax.fori_loop(0, (rem + 1) // 2, body, 0)
            return carry

        lax.fori_loop(0, lax.div(nch + MCH - 1, jnp.int32(MCH)), macro, 0)

    @pl.when(c == 0)
    def _():
        run(xs0_hbm)

    @pl.when(c == 1)
    def _():
        run(xs1_hbm)

    plsc.subcore_barrier()

    @pl.when(c == 0)
    def _():
        pltpu.sync_copy(acc.at[pl.ds(s * STRIPE, STRIPE)],
                        out_hbm.at[pl.ds(s * STRIPE, STRIPE), pl.ds(0, 128)])

    @pl.when(c == 1)
    def _():
        pltpu.sync_copy(acc.at[pl.ds(s * STRIPE, STRIPE)],
                        out_hbm.at[pl.ds(s * STRIPE, STRIPE), pl.ds(128, 128)])


_prop_f = pl.kernel(
    _prop_f_body,
    mesh=_MESH,
    out_type=jax.ShapeDtypeStruct((NPAD, NINP), jnp.float32),
    scratch_types=[
        pltpu.VMEM((MCH, 128), jnp.int32),
        pltpu.VMEM((MCH, 128), jnp.int32),
        pltpu.VMEM((16, 16), jnp.int32),
        pltpu.VMEM((128, 128), jnp.float32),
        pltpu.VMEM((128, 128), jnp.float32),
        pltpu.VMEM_SHARED((NPAD, 128), jnp.float32),
        pltpu.SemaphoreType.DMA,
        pltpu.SemaphoreType.DMA,
    ],
    compiler_params=_SC_PARAMS,
)


def _lookup_body(tok_hbm, s2_hbm, dinv_hbm, b2_hbm, out_hbm,
                 ibuf, dtab, b2v, gbuf, obuf, sem):
    c = lax.axis_index("c")
    s = lax.axis_index("s")
    w = s * 2 + c
    pltpu.sync_copy(tok_hbm.at[w], ibuf)
    pltpu.sync_copy(dinv_hbm, dtab)
    pltpu.sync_copy(b2_hbm, b2v)
    pltpu.async_copy(s2_hbm.at[ibuf], gbuf, sem).wait()
    dv0 = plsc.load_gather(dtab, [ibuf[pl.ds(0, 16)]])
    dv1 = plsc.load_gather(dtab, [ibuf[pl.ds(16, 16)]])
    riota = jnp.arange(16, dtype=jnp.int32)

    def body(f, carry):
        fs = jnp.full((16,), f, dtype=jnp.int32)
        bv = plsc.load_gather(b2v, [fs])
        v0 = plsc.load_gather(gbuf, [riota, fs]) * dv0 + bv
        v1 = plsc.load_gather(gbuf, [riota + 16, fs]) * dv1 + bv
        plsc.store_scatter(obuf, [riota, fs], v0)
        plsc.store_scatter(obuf, [riota + 16, fs], v1)
        return carry

    lax.fori_loop(0, NINP, body, 0)
    pltpu.sync_copy(obuf, out_hbm.at[pl.ds(w * 32, 32)])


_lookup = pl.kernel(
    _lookup_body,
    mesh=_MESH,
    out_type=jax.ShapeDtypeStruct((1024, NINP), jnp.float32),
    scratch_types=[
        pltpu.VMEM((32,), jnp.int32),
        pltpu.VMEM((NPAD,), jnp.float32),
        pltpu.VMEM((NINP,), jnp.float32),
        pltpu.VMEM((32, NINP), jnp.float32),
        pltpu.VMEM((32, NINP), jnp.float32),
        pltpu.SemaphoreType.DMA,
    ],
    compiler_params=_SC_PARAMS,
)


def _scale_body(deg_ref, emb_ref, dinv_ref, xs0_ref, xs1_ref):
    deg = deg_ref[...]
    dv = jnp.where(deg > 0, lax.rsqrt(deg), 0.0)
    dinv_ref[...] = dv
    e = emb_ref[...]
    xs0_ref[...] = e[:, :128] * dv
    xs1_ref[...] = e[:, 128:] * dv


_scale = pl.pallas_call(
    _scale_body,
    grid=(NPAD // BLK,),
    in_specs=[
        pl.BlockSpec((BLK, 1), lambda i: (i, 0)),
        pl.BlockSpec((BLK, NINP), lambda i: (i, 0)),
    ],
    out_specs=[
        pl.BlockSpec((BLK, 1), lambda i: (i, 0)),
        pl.BlockSpec((BLK, 128), lambda i: (i, 0)),
        pl.BlockSpec((BLK, 128), lambda i: (i, 0)),
    ],
    out_shape=[
        jax.ShapeDtypeStruct((NPAD, 1), jnp.float32),
        jax.ShapeDtypeStruct((NPAD, 128), jnp.float32),
        jax.ShapeDtypeStruct((NPAD, 128), jnp.float32),
    ],
)


def _mid_body(s1_ref, dinv_ref, w1_ref, b1_ref, w2_ref, xs20_ref, xs21_ref):
    i = pl.program_id(0)
    dv = dinv_ref[...]
    u = s1_ref[...] * dv
    h = jnp.dot(u, w1_ref[...], preferred_element_type=jnp.float32,
                precision=lax.Precision.HIGHEST) + b1_ref[...]
    y = jnp.dot(h, w2_ref[...], preferred_element_type=jnp.float32,
                precision=lax.Precision.HIGHEST) * dv
    rid = i * BLK + lax.broadcasted_iota(jnp.int32, (BLK, 1), 0)
    y = jnp.where(rid < NTOK, y, 0.0)
    xs20_ref[...] = y[:, :128]
    xs21_ref[...] = y[:, 128:]


_mid = pl.pallas_call(
    _mid_body,
    grid=(NPAD // BLK,),
    in_specs=[
        pl.BlockSpec((BLK, NINP), lambda i: (i, 0)),
        pl.BlockSpec((BLK, 1), lambda i: (i, 0)),
        pl.BlockSpec((NINP, NHID), lambda i: (0, 0)),
        pl.BlockSpec((1, NHID), lambda i: (0, 0)),
        pl.BlockSpec((NHID, NINP), lambda i: (0, 0)),
    ],
    out_specs=[
        pl.BlockSpec((BLK, 128), lambda i: (i, 0)),
        pl.BlockSpec((BLK, 128), lambda i: (i, 0)),
    ],
    out_shape=[
        jax.ShapeDtypeStruct((NPAD, 128), jnp.float32),
        jax.ShapeDtypeStruct((NPAD, 128), jnp.float32),
    ],
)


def kernel(input, input_timestamp, edge_index, emb, W1, b1, W2, b2):
    f32, i32 = jnp.float32, jnp.int32
    ei = edge_index.astype(i32)
    loops = jnp.arange(NTOK, dtype=i32)
    rows = jnp.concatenate([ei[0], loops,
                            jnp.full((EP - NE - NTOK,), PADROW, i32)])
    cols = jnp.concatenate([ei[1], loops,
                            jnp.full((EP - NE - NTOK,), PADROW, i32)])
    rows = rows.reshape(EP // 16, 16).T
    cols = cols.reshape(EP // 16, 16).T
    rows_h = rows.reshape(16, NCH // MCH, MCH, 128)
    cols_h = cols.reshape(16, NCH // MCH, MCH, 128)
    rows_h3 = rows.reshape(16, NCH, 128)
    cols_h3 = cols.reshape(16, NCH, 128)
    cols_deg = jnp.concatenate(
        [ei[1], jnp.full((DEG_EP - NE,), PADROW, i32)]).reshape(16, NCH_DEG, 128)
    ones128 = jnp.ones((128,), f32)
    init640 = jnp.ones((STRIPE,), f32)
    zeros640 = jnp.zeros((STRIPE, 128), f32)
    zbm = jnp.zeros((NPAD,), i32)
    emb_pad = jnp.pad(emb, ((0, NPAD - NTOK), (0, 0)))
    flat = input.reshape(-1).astype(i32)
    tok = jnp.concatenate([flat, jnp.zeros((1024 - flat.shape[0],), i32)])

    deg, crows, ccols, counts = _plan(cols_deg, rows_h3, cols_h3, tok,
                                      ones128, init640, zbm)
    dinv, xs0, xs1 = _scale(deg.reshape(NPAD, 1), emb_pad)
    s1 = _prop(rows_h, cols_h, xs0, xs1, zeros640)
    xs20, xs21 = _mid(s1, dinv, W1, b1.reshape(1, NHID), W2)
    s2 = _prop_f(crows.reshape(16, NCH // MCH, MCH, 128),
                 ccols.reshape(16, NCH // MCH, MCH, 128),
                 counts, xs20, xs21, zeros640)

    out = _lookup(tok.reshape(32, 32), s2, dinv.reshape(-1), b2)
    return out[:flat.shape[0]].reshape(input.shape[0], input.shape[1], NINP)
